# Initial kernel scaffold; baseline (speedup 1.0000x reference)
import jax
import jax.numpy as jnp
from jax.experimental import pallas as pl


def kernel(x_gene, x_cell, W1_gg, b1_gg, W1_rev, b1_rev, W1_cc, b1_cc, W2_gg, b2_gg, W2_rev, b2_rev, W2_cc, b2_cc, edge_index_gg, edge_index_gg_rev, edge_index_cc, edge_label_index):
    raise NotImplementedError("write your pallas kernel here")


# speedup vs baseline: 5.4955x; 5.4955x over previous
import functools

import jax
import jax.numpy as jnp
from jax import lax
from jax.experimental import pallas as pl
from jax.experimental.pallas import tpu as pltpu
from jax.experimental.pallas import tpu_sc as plsc

N = 10000
NPAD = 10240
D_IN = 128
H1 = 256
H2 = 128
E = 320000
E_LBL = 100000

NS = 16
NC = 2
K = 128
NCHUNK = 160
NCHUNK2 = 80
NCHUNK_L = 25
EPAD_L = NS * NC * K * NCHUNK_L
ROWS_PER_TILE = NPAD // NS
EDGES_PER_TILE = NCHUNK * K

_MESH = plsc.VectorSubcoreMesh(core_axis_name="c", subcore_axis_name="s")


def _deg_body(dst_cat, zeros1, hist, idx_v, acc_t, sem):
    c = lax.axis_index("c")
    s = lax.axis_index("s")
    w = c * NS + s
    pltpu.sync_copy(zeros1, acc_t)
    pltpu.sync_copy(dst_cat.at[w], idx_v)
    ones = jnp.ones((16,), jnp.float32)

    def sub(k, carry):
        iv = idx_v[pl.ds(k * 16, 16)]
        plsc.addupdate_scatter(acc_t, [iv], ones)
        return carry

    lax.fori_loop(0, EDGES_PER_TILE // 16, sub, 0)
    pltpu.sync_copy(acc_t, hist.at[c, s])


_degrees = pl.kernel(
    _deg_body,
    mesh=_MESH,
    compiler_params=pltpu.CompilerParams(needs_layout_passes=False),
    out_type=jax.ShapeDtypeStruct((NC, NS, NPAD), jnp.float32),
    scratch_types=[pltpu.VMEM((EDGES_PER_TILE,), jnp.int32),
                   pltpu.VMEM((NPAD,), jnp.float32),
                   pltpu.SemaphoreType.DMA],
)


def _scatter1_body(y_cat, src_t, dst_t, zeros, agg, src_v, dst_v, rows_v, acc, gsem):
    c = lax.axis_index("c")
    s = lax.axis_index("s")
    w = c * NS + s
    rows = pl.ds(s * ROWS_PER_TILE, ROWS_PER_TILE)
    pltpu.sync_copy(zeros.at[rows], acc.at[rows])
    plsc.subcore_barrier()

    def chunk(j, carry):
        pltpu.sync_copy(src_t.at[w, j], src_v)
        pltpu.sync_copy(dst_t.at[s, j], dst_v)
        pltpu.async_copy(y_cat.at[src_v], rows_v, gsem).wait()
        pltpu.sync_copy(rows_v, acc.at[dst_v], add=True)
        return carry

    lax.fori_loop(0, NCHUNK, chunk, 0)
    plsc.subcore_barrier()
    pltpu.sync_copy(acc.at[rows], agg.at[c, rows])


_scatter_h1 = pl.kernel(
    _scatter1_body,
    mesh=_MESH,
    out_type=jax.ShapeDtypeStruct((NC, NPAD, H1 // 2), jnp.float32),
    scratch_types=[pltpu.VMEM((K,), jnp.int32),
                   pltpu.VMEM((K,), jnp.int32),
                   pltpu.VMEM((K, H1 // 2), jnp.float32),
                   pltpu.VMEM_SHARED((NPAD, H1 // 2), jnp.float32),
                   pltpu.SemaphoreType.DMA],
)


def _scatter2_body(y2, src_t, dst_t, zeros, part, src_v, dst_v, rows_v, acc, gsem):
    c = lax.axis_index("c")
    s = lax.axis_index("s")
    w = s * NC + c
    rows = pl.ds(s * ROWS_PER_TILE, ROWS_PER_TILE)
    pltpu.sync_copy(zeros.at[rows], acc.at[rows])
    plsc.subcore_barrier()

    def chunk(j, carry):
        pltpu.sync_copy(src_t.at[w, j], src_v)
        pltpu.sync_copy(dst_t.at[w, j], dst_v)
        pltpu.async_copy(y2.at[src_v], rows_v, gsem).wait()
        pltpu.sync_copy(rows_v, acc.at[dst_v], add=True)
        return carry

    lax.fori_loop(0, NCHUNK2, chunk, 0)
    plsc.subcore_barrier()
    pltpu.sync_copy(acc.at[rows], part.at[c, rows])


_scatter_h2 = pl.kernel(
    _scatter2_body,
    mesh=_MESH,
    out_type=jax.ShapeDtypeStruct((NC, NPAD, H2), jnp.float32),
    scratch_types=[pltpu.VMEM((K,), jnp.int32),
                   pltpu.VMEM((K,), jnp.int32),
                   pltpu.VMEM((K, H2), jnp.float32),
                   pltpu.VMEM_SHARED((NPAD, H2), jnp.float32),
                   pltpu.SemaphoreType.DMA],
)

LBL_PER_TILE = NCHUNK_L * K
LBL_ROWS = LBL_PER_TILE // 8


def _label_body(g2, l0_t, l1_t, pred, i0_v, i1_v, r0_v, r1_v, out_v, s0, s1):
    c = lax.axis_index("c")
    s = lax.axis_index("s")
    w = s * NC + c

    def chunk(j, carry):
        pltpu.sync_copy(l0_t.at[w, j], i0_v)
        pltpu.sync_copy(l1_t.at[w, j], i1_v)
        cp0 = pltpu.async_copy(g2.at[i0_v], r0_v, s0)
        cp1 = pltpu.async_copy(g2.at[i1_v], r1_v, s1)
        cp0.wait()
        cp1.wait()

        def edge(e, carry2):
            acc = r0_v[e, pl.ds(0, 16)] * r1_v[e, pl.ds(0, 16)]
            for k in range(1, H2 // 16):
                acc = acc + r0_v[e, pl.ds(k * 16, 16)] * r1_v[e, pl.ds(k * 16, 16)]
            out_v[j * (K // 8) + e // 8, pl.ds((e % 8) * 16, 16)] = acc
            return carry2

        lax.fori_loop(0, K, edge, 0)
        return carry

    lax.fori_loop(0, NCHUNK_L, chunk, 0)
    pltpu.sync_copy(out_v, pred.at[pl.ds(w * LBL_ROWS, LBL_ROWS)])


_label_dot = pl.kernel(
    _label_body,
    mesh=_MESH,
    out_type=jax.ShapeDtypeStruct((EPAD_L // 8, K), jnp.float32),
    scratch_types=[pltpu.VMEM((K,), jnp.int32),
                   pltpu.VMEM((K,), jnp.int32),
                   pltpu.VMEM((K, H2), jnp.float32),
                   pltpu.VMEM((K, H2), jnp.float32),
                   pltpu.VMEM((LBL_ROWS, K), jnp.float32),
                   pltpu.SemaphoreType.DMA,
                   pltpu.SemaphoreType.DMA],
)


_RB = 512
_GRID = NPAD // _RB
_CB = 1024


def _tc0_body(hgg, hrev, dgg, drev):
    dgg[...] = lax.rsqrt(jnp.sum(hgg[...], axis=0, keepdims=True) + 1.0)
    drev[...] = lax.rsqrt(jnp.sum(hrev[...], axis=0, keepdims=True) + 1.0)


_tc0 = pl.pallas_call(
    _tc0_body,
    out_shape=[jax.ShapeDtypeStruct((1, NPAD), jnp.float32)] * 2,
)


def _tc1_body(x, w_gg, w_rev, dgg, drev, ygg_lo, ygg_hi, yrev_lo, yrev_hi):
    y = jnp.dot(x[...], w_gg[...], preferred_element_type=jnp.float32) * dgg[...]
    ygg_lo[...] = y[:, :H1 // 2]
    ygg_hi[...] = y[:, H1 // 2:]
    y = jnp.dot(x[...], w_rev[...], preferred_element_type=jnp.float32) * drev[...]
    yrev_lo[...] = y[:, :H1 // 2]
    yrev_hi[...] = y[:, H1 // 2:]


def _row_spec(w):
    return pl.BlockSpec((_RB, w), lambda i: (i, 0))


def _full_spec(h, w):
    return pl.BlockSpec((h, w), lambda i: (0, 0))


_tc1 = pl.pallas_call(
    _tc1_body,
    grid=(_GRID,),
    in_specs=[_row_spec(D_IN), _full_spec(D_IN, H1), _full_spec(D_IN, H1),
              _row_spec(1), _row_spec(1)],
    out_specs=[_row_spec(H1 // 2)] * 4,
    out_shape=[jax.ShapeDtypeStruct((NPAD, H1 // 2), jnp.float32)] * 4,
)


def _tc2_body(agg_gg_lo, agg_gg_hi, agg_rev_lo, agg_rev_hi,
              ygg_lo, ygg_hi, yrev_lo, yrev_hi, dgg, drev,
              b_gg, b_rev, w2_gg, w2_rev, y2gg, y2rev):
    agg_gg = jnp.concatenate([agg_gg_lo[...] + ygg_lo[...],
                              agg_gg_hi[...] + ygg_hi[...]], axis=1)
    agg_rev = jnp.concatenate([agg_rev_lo[...] + yrev_lo[...],
                               agg_rev_hi[...] + yrev_hi[...]], axis=1)
    di_gg = dgg[...]
    di_rev = drev[...]
    g = jax.nn.relu(di_gg * agg_gg + b_gg[...] + di_rev * agg_rev + b_rev[...])
    y2gg[...] = jnp.dot(g, w2_gg[...], preferred_element_type=jnp.float32) * di_gg
    y2rev[...] = jnp.dot(g, w2_rev[...], preferred_element_type=jnp.float32) * di_rev


_tc2 = pl.pallas_call(
    _tc2_body,
    grid=(_GRID,),
    in_specs=[_row_spec(H1 // 2)] * 8 + [_row_spec(1)] * 2
             + [_full_spec(1, H1)] * 2 + [_full_spec(H1, H2)] * 2,
    out_specs=[_row_spec(H2)] * 2,
    out_shape=[jax.ShapeDtypeStruct((NPAD, H2), jnp.float32)] * 2,
)


def _tc3_body(agg_gg_p0, agg_gg_p1, agg_rev_p0, agg_rev_p1,
              y2gg, y2rev, dgg, drev, b_gg, b_rev, g2):
    a_gg = agg_gg_p0[...] + agg_gg_p1[...] + y2gg[...]
    a_rev = agg_rev_p0[...] + agg_rev_p1[...] + y2rev[...]
    g2[...] = (dgg[...] * a_gg + b_gg[...] + drev[...] * a_rev + b_rev[...])


_tc3 = pl.pallas_call(
    _tc3_body,
    grid=(_GRID,),
    in_specs=[_row_spec(H2)] * 6 + [_row_spec(1)] * 2
             + [_full_spec(1, H2)] * 2,
    out_specs=_row_spec(H2),
    out_shape=jax.ShapeDtypeStruct((NPAD, H2), jnp.float32),
)


def _tc4_body(p16, sel, pred):
    pred[...] = jnp.dot(p16[...], sel[...], preferred_element_type=jnp.float32)


_LB = 3200

_tc4 = pl.pallas_call(
    _tc4_body,
    grid=(EPAD_L // 8 // _LB,),
    in_specs=[pl.BlockSpec((_LB, K), lambda i: (i, 0)), _full_spec(K, 8)],
    out_specs=pl.BlockSpec((_LB, 8), lambda i: (i, 0)),
    out_shape=jax.ShapeDtypeStruct((EPAD_L // 8, 8), jnp.float32),
)


def _tile_edges(idx, nway, nchunk):
    pad = nway * nchunk * K - idx.shape[0]
    idx = jnp.concatenate([idx, jnp.full((pad,), N, dtype=jnp.int32)])
    return idx.reshape(nway, nchunk, K)


def kernel(x_gene, x_cell, W1_gg, b1_gg, W1_rev, b1_rev, W1_cc, b1_cc,
           W2_gg, b2_gg, W2_rev, b2_rev, W2_cc, b2_cc,
           edge_index_gg, edge_index_gg_rev, edge_index_cc, edge_label_index):
    x = jnp.pad(x_gene, ((0, NPAD - N), (0, 0)))
    src_gg = _tile_edges(edge_index_gg[0], NS, NCHUNK)
    dst_gg = _tile_edges(edge_index_gg[1], NS, NCHUNK)
    src_rev = _tile_edges(edge_index_gg_rev[0], NS, NCHUNK)
    dst_rev = _tile_edges(edge_index_gg_rev[1], NS, NCHUNK)
    src_gg2 = _tile_edges(edge_index_gg[0], NS * NC, NCHUNK2)
    dst_gg2 = _tile_edges(edge_index_gg[1], NS * NC, NCHUNK2)
    src_rev2 = _tile_edges(edge_index_gg_rev[0], NS * NC, NCHUNK2)
    dst_rev2 = _tile_edges(edge_index_gg_rev[1], NS * NC, NCHUNK2)
    l0 = _tile_edges(edge_label_index[0], NS * NC, NCHUNK_L)
    l1 = _tile_edges(edge_label_index[1], NS * NC, NCHUNK_L)

    zeros1 = jnp.zeros((NPAD,), jnp.float32)
    z128 = jnp.zeros((NPAD, H1 // 2), jnp.float32)

    dst_cat = jnp.concatenate([dst_gg, dst_rev], axis=0).reshape(NC * NS, -1)
    hist = _degrees(dst_cat, zeros1)
    dinv_gg, dinv_rev = _tc0(hist[0], hist[1])
    dinv_gg = dinv_gg.reshape(NPAD, 1)
    dinv_rev = dinv_rev.reshape(NPAD, 1)

    ygg_lo, ygg_hi, yrev_lo, yrev_hi = _tc1(x, W1_gg, W1_rev, dinv_gg, dinv_rev)

    src_gg_cat = jnp.concatenate([src_gg, src_gg + NPAD], axis=0)
    src_rev_cat = jnp.concatenate([src_rev, src_rev + NPAD], axis=0)
    ygg_cat = jnp.concatenate([ygg_lo, ygg_hi], axis=0)
    yrev_cat = jnp.concatenate([yrev_lo, yrev_hi], axis=0)
    agg_gg = _scatter_h1(ygg_cat, src_gg_cat, dst_gg, z128)
    agg_rev = _scatter_h1(yrev_cat, src_rev_cat, dst_rev, z128)

    y2gg, y2rev = _tc2(
        agg_gg[0], agg_gg[1], agg_rev[0], agg_rev[1],
        ygg_lo, ygg_hi, yrev_lo, yrev_hi, dinv_gg, dinv_rev,
        b1_gg.reshape(1, H1), b1_rev.reshape(1, H1), W2_gg, W2_rev)

    agg2_gg = _scatter_h2(y2gg, src_gg2, dst_gg2, z128)
    agg2_rev = _scatter_h2(y2rev, src_rev2, dst_rev2, z128)

    g2 = _tc3(agg2_gg[0], agg2_gg[1], agg2_rev[0], agg2_rev[1],
              y2gg, y2rev, dinv_gg, dinv_rev,
              b2_gg.reshape(1, H2), b2_rev.reshape(1, H2))

    pred16 = _label_dot(g2, l0, l1)
    sel = (jnp.arange(K)[:, None] // 16 == jnp.arange(8)[None, :]).astype(jnp.float32)
    pred = _tc4(pred16, sel)
    return pred.reshape(EPAD_L)[:E_LBL]


# speedup vs baseline: 6.4664x; 1.1767x over previous
import functools

import jax
import jax.numpy as jnp
from jax import lax
from jax.experimental import pallas as pl
from jax.experimental.pallas import tpu as pltpu
from jax.experimental.pallas import tpu_sc as plsc

N = 10000
NPAD = 10240
D_IN = 128
H1 = 256
H2 = 128
E = 320000
E_LBL = 100000

NS = 16
NC = 2
K = 128
NCHUNK = 160
NCHUNK2 = 80
NCHUNK_L = 26
EPAD_L = NS * NC * K * NCHUNK_L
ROWS_PER_TILE = NPAD // NS
EDGES_PER_TILE = NCHUNK * K

_MESH = plsc.VectorSubcoreMesh(core_axis_name="c", subcore_axis_name="s")


def _deg_body(dst_cat, zeros1, hist, idx_v, acc_t, sem):
    c = lax.axis_index("c")
    s = lax.axis_index("s")
    w = c * NS + s
    pltpu.sync_copy(zeros1, acc_t)
    pltpu.sync_copy(dst_cat.at[w], idx_v)
    ones = jnp.ones((16,), jnp.float32)

    def sub(k, carry):
        iv = idx_v[pl.ds(k * 16, 16)]
        plsc.addupdate_scatter(acc_t, [iv], ones)
        return carry

    lax.fori_loop(0, EDGES_PER_TILE // 16, sub, 0)
    pltpu.sync_copy(acc_t, hist.at[c, s])


_degrees = pl.kernel(
    _deg_body,
    mesh=_MESH,
    compiler_params=pltpu.CompilerParams(needs_layout_passes=False),
    out_type=jax.ShapeDtypeStruct((NC, NS, NPAD), jnp.float32),
    scratch_types=[pltpu.VMEM((EDGES_PER_TILE,), jnp.int32),
                   pltpu.VMEM((NPAD,), jnp.float32),
                   pltpu.SemaphoreType.DMA],
)


def _pipelined_scatter(y, src_t, dst_t, acc, w_src, w_dst, nchunk,
                       src_a, dst_a, rows_a, ga, sa,
                       src_b, dst_b, rows_b, gb, sb):
    def stage(j, src_v, dst_v):
        pltpu.sync_copy(src_t.at[w_src, j], src_v)
        pltpu.sync_copy(dst_t.at[w_dst, j], dst_v)

    stage(0, src_a, dst_a)
    pltpu.async_copy(y.at[src_a], rows_a, ga)
    nj2 = nchunk // 2

    def body(jj, carry):
        j0 = 2 * jj

        @pl.when(jj > 0)
        def _():
            pltpu.make_async_copy(rows_b, acc.at[dst_b], sb).wait()

        stage(j0 + 1, src_b, dst_b)
        pltpu.async_copy(y.at[src_b], rows_b, gb)
        pltpu.make_async_copy(y.at[src_a], rows_a, ga).wait()
        pltpu.async_copy(rows_a, acc.at[dst_a], sa, add=True)

        @pl.when(jj + 1 < nj2)
        def _():
            pltpu.make_async_copy(rows_a, acc.at[dst_a], sa).wait()
            stage(j0 + 2, src_a, dst_a)
            pltpu.async_copy(y.at[src_a], rows_a, ga)

        pltpu.make_async_copy(y.at[src_b], rows_b, gb).wait()
        pltpu.async_copy(rows_b, acc.at[dst_b], sb, add=True)
        return carry

    lax.fori_loop(0, nj2, body, 0)
    pltpu.make_async_copy(rows_a, acc.at[dst_a], sa).wait()
    pltpu.make_async_copy(rows_b, acc.at[dst_b], sb).wait()


def _scatter_scratch(dh):
    return [pltpu.VMEM((K,), jnp.int32),
            pltpu.VMEM((K,), jnp.int32),
            pltpu.VMEM((K, dh), jnp.float32),
            pltpu.VMEM((K,), jnp.int32),
            pltpu.VMEM((K,), jnp.int32),
            pltpu.VMEM((K, dh), jnp.float32),
            pltpu.VMEM_SHARED((NPAD, dh), jnp.float32),
            pltpu.SemaphoreType.DMA,
            pltpu.SemaphoreType.DMA,
            pltpu.SemaphoreType.DMA,
            pltpu.SemaphoreType.DMA]


def _scatter1_body(y_cat, src_t, dst_t, zeros, agg,
                   src_a, dst_a, rows_a, src_b, dst_b, rows_b,
                   acc, ga, sa, gb, sb):
    c = lax.axis_index("c")
    s = lax.axis_index("s")
    w = c * NS + s
    rows = pl.ds(s * ROWS_PER_TILE, ROWS_PER_TILE)
    pltpu.sync_copy(zeros.at[rows], acc.at[rows])
    plsc.subcore_barrier()
    _pipelined_scatter(y_cat, src_t, dst_t, acc, w, s, NCHUNK,
                       src_a, dst_a, rows_a, ga, sa,
                       src_b, dst_b, rows_b, gb, sb)
    plsc.subcore_barrier()
    pltpu.sync_copy(acc.at[rows], agg.at[c, rows])


_scatter_h1 = pl.kernel(
    _scatter1_body,
    mesh=_MESH,
    out_type=jax.ShapeDtypeStruct((NC, NPAD, H1 // 2), jnp.float32),
    scratch_types=_scatter_scratch(H1 // 2),
)


def _scatter2_body(y2, src_t, dst_t, zeros, part,
                   src_a, dst_a, rows_a, src_b, dst_b, rows_b,
                   acc, ga, sa, gb, sb):
    c = lax.axis_index("c")
    s = lax.axis_index("s")
    w = s * NC + c
    rows = pl.ds(s * ROWS_PER_TILE, ROWS_PER_TILE)
    pltpu.sync_copy(zeros.at[rows], acc.at[rows])
    plsc.subcore_barrier()
    _pipelined_scatter(y2, src_t, dst_t, acc, w, w, NCHUNK2,
                       src_a, dst_a, rows_a, ga, sa,
                       src_b, dst_b, rows_b, gb, sb)
    plsc.subcore_barrier()
    pltpu.sync_copy(acc.at[rows], part.at[c, rows])


_scatter_h2 = pl.kernel(
    _scatter2_body,
    mesh=_MESH,
    out_type=jax.ShapeDtypeStruct((NC, NPAD, H2), jnp.float32),
    scratch_types=_scatter_scratch(H2),
)


LBL_PER_TILE = NCHUNK_L * K
LBL_ROWS = LBL_PER_TILE // 8


def _label_body(g2, l0_t, l1_t, pred, i0a, i1a, r0a, r1a, i0b, i1b, r0b, r1b,
                out_v, s0a, s1a, s0b, s1b):
    c = lax.axis_index("c")
    s = lax.axis_index("s")
    w = s * NC + c

    def stage(j, i0_v, i1_v, r0_v, r1_v, sg0, sg1):
        pltpu.sync_copy(l0_t.at[w, j], i0_v)
        pltpu.sync_copy(l1_t.at[w, j], i1_v)
        pltpu.async_copy(g2.at[i0_v], r0_v, sg0)
        pltpu.async_copy(g2.at[i1_v], r1_v, sg1)

    def compute(j, i0_v, i1_v, r0_v, r1_v, sg0, sg1):
        pltpu.make_async_copy(g2.at[i0_v], r0_v, sg0).wait()
        pltpu.make_async_copy(g2.at[i1_v], r1_v, sg1).wait()

        def edge(e, carry2):
            acc = r0_v[e, pl.ds(0, 16)] * r1_v[e, pl.ds(0, 16)]
            for k in range(1, H2 // 16):
                acc = acc + r0_v[e, pl.ds(k * 16, 16)] * r1_v[e, pl.ds(k * 16, 16)]
            out_v[j * (K // 8) + e // 8, pl.ds((e % 8) * 16, 16)] = acc
            return carry2

        lax.fori_loop(0, K, edge, 0)

    stage(0, i0a, i1a, r0a, r1a, s0a, s1a)

    def body(jj, carry):
        j0 = 2 * jj
        stage(j0 + 1, i0b, i1b, r0b, r1b, s0b, s1b)
        compute(j0, i0a, i1a, r0a, r1a, s0a, s1a)

        @pl.when(jj + 1 < NCHUNK_L // 2)
        def _():
            stage(j0 + 2, i0a, i1a, r0a, r1a, s0a, s1a)

        compute(j0 + 1, i0b, i1b, r0b, r1b, s0b, s1b)
        return carry

    lax.fori_loop(0, NCHUNK_L // 2, body, 0)
    pltpu.sync_copy(out_v, pred.at[pl.ds(w * LBL_ROWS, LBL_ROWS)])


_label_dot = pl.kernel(
    _label_body,
    mesh=_MESH,
    out_type=jax.ShapeDtypeStruct((EPAD_L // 8, K), jnp.float32),
    scratch_types=[pltpu.VMEM((K,), jnp.int32),
                   pltpu.VMEM((K,), jnp.int32),
                   pltpu.VMEM((K, H2), jnp.float32),
                   pltpu.VMEM((K, H2), jnp.float32),
                   pltpu.VMEM((K,), jnp.int32),
                   pltpu.VMEM((K,), jnp.int32),
                   pltpu.VMEM((K, H2), jnp.float32),
                   pltpu.VMEM((K, H2), jnp.float32),
                   pltpu.VMEM((LBL_ROWS, K), jnp.float32),
                   pltpu.SemaphoreType.DMA,
                   pltpu.SemaphoreType.DMA,
                   pltpu.SemaphoreType.DMA,
                   pltpu.SemaphoreType.DMA],
)


_RB = 512
_GRID = NPAD // _RB
_CB = 1024


def _tc0_body(hgg, hrev, dgg, drev):
    dgg[...] = lax.rsqrt(jnp.sum(hgg[...], axis=0, keepdims=True) + 1.0)
    drev[...] = lax.rsqrt(jnp.sum(hrev[...], axis=0, keepdims=True) + 1.0)


_tc0 = pl.pallas_call(
    _tc0_body,
    out_shape=[jax.ShapeDtypeStruct((1, NPAD), jnp.float32)] * 2,
)


def _tc1_body(x, w_gg, w_rev, dgg, drev, ygg_lo, ygg_hi, yrev_lo, yrev_hi):
    y = jnp.dot(x[...], w_gg[...], preferred_element_type=jnp.float32) * dgg[...]
    ygg_lo[...] = y[:, :H1 // 2]
    ygg_hi[...] = y[:, H1 // 2:]
    y = jnp.dot(x[...], w_rev[...], preferred_element_type=jnp.float32) * drev[...]
    yrev_lo[...] = y[:, :H1 // 2]
    yrev_hi[...] = y[:, H1 // 2:]


def _row_spec(w):
    return pl.BlockSpec((_RB, w), lambda i: (i, 0))


def _full_spec(h, w):
    return pl.BlockSpec((h, w), lambda i: (0, 0))


_tc1 = pl.pallas_call(
    _tc1_body,
    grid=(_GRID,),
    in_specs=[_row_spec(D_IN), _full_spec(D_IN, H1), _full_spec(D_IN, H1),
              _row_spec(1), _row_spec(1)],
    out_specs=[_row_spec(H1 // 2)] * 4,
    out_shape=[jax.ShapeDtypeStruct((NPAD, H1 // 2), jnp.float32)] * 4,
)


def _tc2_body(agg_gg_lo, agg_gg_hi, agg_rev_lo, agg_rev_hi,
              ygg_lo, ygg_hi, yrev_lo, yrev_hi, dgg, drev,
              b_gg, b_rev, w2_gg, w2_rev, y2gg, y2rev):
    agg_gg = jnp.concatenate([agg_gg_lo[...] + ygg_lo[...],
                              agg_gg_hi[...] + ygg_hi[...]], axis=1)
    agg_rev = jnp.concatenate([agg_rev_lo[...] + yrev_lo[...],
                               agg_rev_hi[...] + yrev_hi[...]], axis=1)
    di_gg = dgg[...]
    di_rev = drev[...]
    g = jax.nn.relu(di_gg * agg_gg + b_gg[...] + di_rev * agg_rev + b_rev[...])
    y2gg[...] = jnp.dot(g, w2_gg[...], preferred_element_type=jnp.float32) * di_gg
    y2rev[...] = jnp.dot(g, w2_rev[...], preferred_element_type=jnp.float32) * di_rev


_tc2 = pl.pallas_call(
    _tc2_body,
    grid=(_GRID,),
    in_specs=[_row_spec(H1 // 2)] * 8 + [_row_spec(1)] * 2
             + [_full_spec(1, H1)] * 2 + [_full_spec(H1, H2)] * 2,
    out_specs=[_row_spec(H2)] * 2,
    out_shape=[jax.ShapeDtypeStruct((NPAD, H2), jnp.float32)] * 2,
)


def _tc3_body(agg_gg_p0, agg_gg_p1, agg_rev_p0, agg_rev_p1,
              y2gg, y2rev, dgg, drev, b_gg, b_rev, g2):
    a_gg = agg_gg_p0[...] + agg_gg_p1[...] + y2gg[...]
    a_rev = agg_rev_p0[...] + agg_rev_p1[...] + y2rev[...]
    g2[...] = (dgg[...] * a_gg + b_gg[...] + drev[...] * a_rev + b_rev[...])


_tc3 = pl.pallas_call(
    _tc3_body,
    grid=(_GRID,),
    in_specs=[_row_spec(H2)] * 6 + [_row_spec(1)] * 2
             + [_full_spec(1, H2)] * 2,
    out_specs=_row_spec(H2),
    out_shape=jax.ShapeDtypeStruct((NPAD, H2), jnp.float32),
)


def _tc4_body(p16, sel, pred):
    pred[...] = jnp.dot(p16[...], sel[...], preferred_element_type=jnp.float32)


_LB = 3328

_tc4 = pl.pallas_call(
    _tc4_body,
    grid=(EPAD_L // 8 // _LB,),
    in_specs=[pl.BlockSpec((_LB, K), lambda i: (i, 0)), _full_spec(K, 8)],
    out_specs=pl.BlockSpec((_LB, 8), lambda i: (i, 0)),
    out_shape=jax.ShapeDtypeStruct((EPAD_L // 8, 8), jnp.float32),
)


def _tile_edges(idx, nway, nchunk):
    pad = nway * nchunk * K - idx.shape[0]
    idx = jnp.concatenate([idx, jnp.full((pad,), N, dtype=jnp.int32)])
    return idx.reshape(nway, nchunk, K)


def kernel(x_gene, x_cell, W1_gg, b1_gg, W1_rev, b1_rev, W1_cc, b1_cc,
           W2_gg, b2_gg, W2_rev, b2_rev, W2_cc, b2_cc,
           edge_index_gg, edge_index_gg_rev, edge_index_cc, edge_label_index):
    x = jnp.pad(x_gene, ((0, NPAD - N), (0, 0)))
    src_gg = _tile_edges(edge_index_gg[0], NS, NCHUNK)
    dst_gg = _tile_edges(edge_index_gg[1], NS, NCHUNK)
    src_rev = _tile_edges(edge_index_gg_rev[0], NS, NCHUNK)
    dst_rev = _tile_edges(edge_index_gg_rev[1], NS, NCHUNK)
    src_gg2 = _tile_edges(edge_index_gg[0], NS * NC, NCHUNK2)
    dst_gg2 = _tile_edges(edge_index_gg[1], NS * NC, NCHUNK2)
    src_rev2 = _tile_edges(edge_index_gg_rev[0], NS * NC, NCHUNK2)
    dst_rev2 = _tile_edges(edge_index_gg_rev[1], NS * NC, NCHUNK2)
    l0 = _tile_edges(edge_label_index[0], NS * NC, NCHUNK_L)
    l1 = _tile_edges(edge_label_index[1], NS * NC, NCHUNK_L)

    zeros1 = jnp.zeros((NPAD,), jnp.float32)
    z128 = jnp.zeros((NPAD, H1 // 2), jnp.float32)

    dst_cat = jnp.concatenate([dst_gg, dst_rev], axis=0).reshape(NC * NS, -1)
    hist = _degrees(dst_cat, zeros1)
    dinv_gg, dinv_rev = _tc0(hist[0], hist[1])
    dinv_gg = dinv_gg.reshape(NPAD, 1)
    dinv_rev = dinv_rev.reshape(NPAD, 1)

    ygg_lo, ygg_hi, yrev_lo, yrev_hi = _tc1(x, W1_gg, W1_rev, dinv_gg, dinv_rev)

    src_gg_cat = jnp.concatenate([src_gg, src_gg + NPAD], axis=0)
    src_rev_cat = jnp.concatenate([src_rev, src_rev + NPAD], axis=0)
    ygg_cat = jnp.concatenate([ygg_lo, ygg_hi], axis=0)
    yrev_cat = jnp.concatenate([yrev_lo, yrev_hi], axis=0)
    agg_gg = _scatter_h1(ygg_cat, src_gg_cat, dst_gg, z128)
    agg_rev = _scatter_h1(yrev_cat, src_rev_cat, dst_rev, z128)

    y2gg, y2rev = _tc2(
        agg_gg[0], agg_gg[1], agg_rev[0], agg_rev[1],
        ygg_lo, ygg_hi, yrev_lo, yrev_hi, dinv_gg, dinv_rev,
        b1_gg.reshape(1, H1), b1_rev.reshape(1, H1), W2_gg, W2_rev)

    agg2_gg = _scatter_h2(y2gg, src_gg2, dst_gg2, z128)
    agg2_rev = _scatter_h2(y2rev, src_rev2, dst_rev2, z128)

    g2 = _tc3(agg2_gg[0], agg2_gg[1], agg2_rev[0], agg2_rev[1],
              y2gg, y2rev, dinv_gg, dinv_rev,
              b2_gg.reshape(1, H2), b2_rev.reshape(1, H2))

    pred16 = _label_dot(g2, l0, l1)
    sel = (jnp.arange(K)[:, None] // 16 == jnp.arange(8)[None, :]).astype(jnp.float32)
    pred = _tc4(pred16, sel)
    return pred.reshape(EPAD_L)[:E_LBL]


# speedup vs baseline: 6.5098x; 1.0067x over previous
import functools

import jax
import jax.numpy as jnp
from jax import lax
from jax.experimental import pallas as pl
from jax.experimental.pallas import tpu as pltpu
from jax.experimental.pallas import tpu_sc as plsc

N = 10000
NPAD = 10240
D_IN = 128
H1 = 256
H2 = 128
E = 320000
E_LBL = 100000

NS = 16
NC = 2
K = 128
NCHUNK = 160
NCHUNK2 = 80
NCHUNK_L = 26
EPAD_L = NS * NC * K * NCHUNK_L
ROWS_PER_TILE = NPAD // NS
EDGES_PER_TILE = NCHUNK * K

_MESH = plsc.VectorSubcoreMesh(core_axis_name="c", subcore_axis_name="s")


def _deg_body(dst_cat, zeros1, hist, idx_v, acc_t, sem):
    c = lax.axis_index("c")
    s = lax.axis_index("s")
    w = c * NS + s
    pltpu.sync_copy(zeros1, acc_t)
    pltpu.sync_copy(dst_cat.at[w], idx_v)
    ones = jnp.ones((16,), jnp.float32)

    def sub(k, carry):
        iv = idx_v[pl.ds(k * 16, 16)]
        plsc.addupdate_scatter(acc_t, [iv], ones)
        return carry

    lax.fori_loop(0, EDGES_PER_TILE // 16, sub, 0)
    pltpu.sync_copy(acc_t, hist.at[c, s])


_degrees = pl.kernel(
    _deg_body,
    mesh=_MESH,
    compiler_params=pltpu.CompilerParams(needs_layout_passes=False),
    out_type=jax.ShapeDtypeStruct((NC, NS, NPAD), jnp.float32),
    scratch_types=[pltpu.VMEM((EDGES_PER_TILE,), jnp.int32),
                   pltpu.VMEM((NPAD,), jnp.float32),
                   pltpu.SemaphoreType.DMA],
)


NBI = 8


def _pipelined_scatter(y, src_t, dst_t, acc, w_src, w_dst, nchunk,
                       isa, ida, isb, idb, rows_a, rows_b,
                       ga, sa, gb, sb, ia, ib):
    def astage(b, i_s, i_d, sem):
        pltpu.async_copy(src_t.at[w_src, b], i_s, sem)
        pltpu.async_copy(dst_t.at[w_dst, b], i_d, sem)

    def wstage(i_s, i_d, sem):
        pltpu.make_async_copy(src_t.at[w_src, 0], i_s, sem).wait()
        pltpu.make_async_copy(dst_t.at[w_dst, 0], i_d, sem).wait()

    def inner(i_s, i_d):
        pltpu.async_copy(y.at[i_s.at[0]], rows_a, ga)
        for jj in range(NBI // 2):
            j0 = 2 * jj
            if jj > 0:
                pltpu.make_async_copy(rows_b, acc.at[i_d.at[j0 - 1]], sb).wait()
            pltpu.async_copy(y.at[i_s.at[j0 + 1]], rows_b, gb)
            pltpu.make_async_copy(y.at[i_s.at[j0]], rows_a, ga).wait()
            pltpu.async_copy(rows_a, acc.at[i_d.at[j0]], sa, add=True)
            if jj < NBI // 2 - 1:
                pltpu.make_async_copy(rows_a, acc.at[i_d.at[j0]], sa).wait()
                pltpu.async_copy(y.at[i_s.at[j0 + 2]], rows_a, ga)
            pltpu.make_async_copy(y.at[i_s.at[j0 + 1]], rows_b, gb).wait()
            pltpu.async_copy(rows_b, acc.at[i_d.at[j0 + 1]], sb, add=True)
        pltpu.make_async_copy(rows_a, acc.at[i_d.at[NBI - 2]], sa).wait()
        pltpu.make_async_copy(rows_b, acc.at[i_d.at[NBI - 1]], sb).wait()

    nb2 = nchunk // NBI // 2
    astage(0, isa, ida, ia)
    astage(1, isb, idb, ib)

    def outer(bb, carry):
        wstage(isa, ida, ia)
        inner(isa, ida)

        @pl.when(bb + 1 < nb2)
        def _():
            astage(2 * bb + 2, isa, ida, ia)

        wstage(isb, idb, ib)
        inner(isb, idb)

        @pl.when(bb + 1 < nb2)
        def _():
            astage(2 * bb + 3, isb, idb, ib)

        return carry

    lax.fori_loop(0, nb2, outer, 0)


def _scatter_scratch(dh):
    return [pltpu.VMEM((NBI, K), jnp.int32),
            pltpu.VMEM((NBI, K), jnp.int32),
            pltpu.VMEM((NBI, K), jnp.int32),
            pltpu.VMEM((NBI, K), jnp.int32),
            pltpu.VMEM((K, dh), jnp.float32),
            pltpu.VMEM((K, dh), jnp.float32),
            pltpu.VMEM_SHARED((NPAD, dh), jnp.float32),
            pltpu.SemaphoreType.DMA,
            pltpu.SemaphoreType.DMA,
            pltpu.SemaphoreType.DMA,
            pltpu.SemaphoreType.DMA,
            pltpu.SemaphoreType.DMA,
            pltpu.SemaphoreType.DMA]


def _scatter1_body(y_cat, src_t, dst_t, zeros, agg,
                   isa, ida, isb, idb, rows_a, rows_b,
                   acc, ga, sa, gb, sb, ia, ib):
    c = lax.axis_index("c")
    s = lax.axis_index("s")
    w = c * NS + s
    rows = pl.ds(s * ROWS_PER_TILE, ROWS_PER_TILE)
    pltpu.sync_copy(zeros.at[rows], acc.at[rows])
    plsc.subcore_barrier()
    _pipelined_scatter(y_cat, src_t, dst_t, acc, w, s, NCHUNK,
                       isa, ida, isb, idb, rows_a, rows_b,
                       ga, sa, gb, sb, ia, ib)
    plsc.subcore_barrier()
    pltpu.sync_copy(acc.at[rows], agg.at[c, rows])


_scatter_h1 = pl.kernel(
    _scatter1_body,
    mesh=_MESH,
    out_type=jax.ShapeDtypeStruct((NC, NPAD, H1 // 2), jnp.float32),
    scratch_types=_scatter_scratch(H1 // 2),
)


def _scatter2_body(y2, src_t, dst_t, zeros, part,
                   isa, ida, isb, idb, rows_a, rows_b,
                   acc, ga, sa, gb, sb, ia, ib):
    c = lax.axis_index("c")
    s = lax.axis_index("s")
    w = s * NC + c
    rows = pl.ds(s * ROWS_PER_TILE, ROWS_PER_TILE)
    pltpu.sync_copy(zeros.at[rows], acc.at[rows])
    plsc.subcore_barrier()
    _pipelined_scatter(y2, src_t, dst_t, acc, w, w, NCHUNK2,
                       isa, ida, isb, idb, rows_a, rows_b,
                       ga, sa, gb, sb, ia, ib)
    plsc.subcore_barrier()
    pltpu.sync_copy(acc.at[rows], part.at[c, rows])


_scatter_h2 = pl.kernel(
    _scatter2_body,
    mesh=_MESH,
    out_type=jax.ShapeDtypeStruct((NC, NPAD, H2), jnp.float32),
    scratch_types=_scatter_scratch(H2),
)


LBL_PER_TILE = NCHUNK_L * K
LBL_ROWS = LBL_PER_TILE // 8


def _label_body(g2, l0_t, l1_t, pred, i0a, i1a, r0a, r1a, i0b, i1b, r0b, r1b,
                out_v, s0a, s1a, s0b, s1b):
    c = lax.axis_index("c")
    s = lax.axis_index("s")
    w = s * NC + c

    def stage(j, i0_v, i1_v, r0_v, r1_v, sg0, sg1):
        pltpu.sync_copy(l0_t.at[w, j], i0_v)
        pltpu.sync_copy(l1_t.at[w, j], i1_v)
        pltpu.async_copy(g2.at[i0_v], r0_v, sg0)
        pltpu.async_copy(g2.at[i1_v], r1_v, sg1)

    def compute(j, i0_v, i1_v, r0_v, r1_v, sg0, sg1):
        pltpu.make_async_copy(g2.at[i0_v], r0_v, sg0).wait()
        pltpu.make_async_copy(g2.at[i1_v], r1_v, sg1).wait()

        def edge(e, carry2):
            acc = r0_v[e, pl.ds(0, 16)] * r1_v[e, pl.ds(0, 16)]
            for k in range(1, H2 // 16):
                acc = acc + r0_v[e, pl.ds(k * 16, 16)] * r1_v[e, pl.ds(k * 16, 16)]
            out_v[j * (K // 8) + e // 8, pl.ds((e % 8) * 16, 16)] = acc
            return carry2

        lax.fori_loop(0, K, edge, 0)

    stage(0, i0a, i1a, r0a, r1a, s0a, s1a)

    def body(jj, carry):
        j0 = 2 * jj
        stage(j0 + 1, i0b, i1b, r0b, r1b, s0b, s1b)
        compute(j0, i0a, i1a, r0a, r1a, s0a, s1a)

        @pl.when(jj + 1 < NCHUNK_L // 2)
        def _():
            stage(j0 + 2, i0a, i1a, r0a, r1a, s0a, s1a)

        compute(j0 + 1, i0b, i1b, r0b, r1b, s0b, s1b)
        return carry

    lax.fori_loop(0, NCHUNK_L // 2, body, 0)
    pltpu.sync_copy(out_v, pred.at[pl.ds(w * LBL_ROWS, LBL_ROWS)])


_label_dot = pl.kernel(
    _label_body,
    mesh=_MESH,
    out_type=jax.ShapeDtypeStruct((EPAD_L // 8, K), jnp.float32),
    scratch_types=[pltpu.VMEM((K,), jnp.int32),
                   pltpu.VMEM((K,), jnp.int32),
                   pltpu.VMEM((K, H2), jnp.float32),
                   pltpu.VMEM((K, H2), jnp.float32),
                   pltpu.VMEM((K,), jnp.int32),
                   pltpu.VMEM((K,), jnp.int32),
                   pltpu.VMEM((K, H2), jnp.float32),
                   pltpu.VMEM((K, H2), jnp.float32),
                   pltpu.VMEM((LBL_ROWS, K), jnp.float32),
                   pltpu.SemaphoreType.DMA,
                   pltpu.SemaphoreType.DMA,
                   pltpu.SemaphoreType.DMA,
                   pltpu.SemaphoreType.DMA],
)


_RB = 512
_GRID = NPAD // _RB
_CB = 1024


def _tc0_body(hgg, hrev, dgg, drev):
    dgg[...] = lax.rsqrt(jnp.sum(hgg[...], axis=0, keepdims=True) + 1.0)
    drev[...] = lax.rsqrt(jnp.sum(hrev[...], axis=0, keepdims=True) + 1.0)


_tc0 = pl.pallas_call(
    _tc0_body,
    out_shape=[jax.ShapeDtypeStruct((1, NPAD), jnp.float32)] * 2,
)


def _tc1_body(x, w_gg, w_rev, dgg, drev, ygg_lo, ygg_hi, yrev_lo, yrev_hi):
    y = jnp.dot(x[...], w_gg[...], preferred_element_type=jnp.float32) * dgg[...]
    ygg_lo[...] = y[:, :H1 // 2]
    ygg_hi[...] = y[:, H1 // 2:]
    y = jnp.dot(x[...], w_rev[...], preferred_element_type=jnp.float32) * drev[...]
    yrev_lo[...] = y[:, :H1 // 2]
    yrev_hi[...] = y[:, H1 // 2:]


def _row_spec(w):
    return pl.BlockSpec((_RB, w), lambda i: (i, 0))


def _full_spec(h, w):
    return pl.BlockSpec((h, w), lambda i: (0, 0))


_tc1 = pl.pallas_call(
    _tc1_body,
    grid=(_GRID,),
    in_specs=[_row_spec(D_IN), _full_spec(D_IN, H1), _full_spec(D_IN, H1),
              _row_spec(1), _row_spec(1)],
    out_specs=[_row_spec(H1 // 2)] * 4,
    out_shape=[jax.ShapeDtypeStruct((NPAD, H1 // 2), jnp.float32)] * 4,
)


def _tc2_body(agg_gg_lo, agg_gg_hi, agg_rev_lo, agg_rev_hi,
              ygg_lo, ygg_hi, yrev_lo, yrev_hi, dgg, drev,
              b_gg, b_rev, w2_gg, w2_rev, y2gg, y2rev):
    agg_gg = jnp.concatenate([agg_gg_lo[...] + ygg_lo[...],
                              agg_gg_hi[...] + ygg_hi[...]], axis=1)
    agg_rev = jnp.concatenate([agg_rev_lo[...] + yrev_lo[...],
                               agg_rev_hi[...] + yrev_hi[...]], axis=1)
    di_gg = dgg[...]
    di_rev = drev[...]
    g = jax.nn.relu(di_gg * agg_gg + b_gg[...] + di_rev * agg_rev + b_rev[...])
    y2gg[...] = jnp.dot(g, w2_gg[...], preferred_element_type=jnp.float32) * di_gg
    y2rev[...] = jnp.dot(g, w2_rev[...], preferred_element_type=jnp.float32) * di_rev


_tc2 = pl.pallas_call(
    _tc2_body,
    grid=(_GRID,),
    in_specs=[_row_spec(H1 // 2)] * 8 + [_row_spec(1)] * 2
             + [_full_spec(1, H1)] * 2 + [_full_spec(H1, H2)] * 2,
    out_specs=[_row_spec(H2)] * 2,
    out_shape=[jax.ShapeDtypeStruct((NPAD, H2), jnp.float32)] * 2,
)


def _tc3_body(agg_gg_p0, agg_gg_p1, agg_rev_p0, agg_rev_p1,
              y2gg, y2rev, dgg, drev, b_gg, b_rev, g2):
    a_gg = agg_gg_p0[...] + agg_gg_p1[...] + y2gg[...]
    a_rev = agg_rev_p0[...] + agg_rev_p1[...] + y2rev[...]
    g2[...] = (dgg[...] * a_gg + b_gg[...] + drev[...] * a_rev + b_rev[...])


_tc3 = pl.pallas_call(
    _tc3_body,
    grid=(_GRID,),
    in_specs=[_row_spec(H2)] * 6 + [_row_spec(1)] * 2
             + [_full_spec(1, H2)] * 2,
    out_specs=_row_spec(H2),
    out_shape=jax.ShapeDtypeStruct((NPAD, H2), jnp.float32),
)


def _tc4_body(p16, sel, pred):
    pred[...] = jnp.dot(p16[...], sel[...], preferred_element_type=jnp.float32)


_LB = 3328

_tc4 = pl.pallas_call(
    _tc4_body,
    grid=(EPAD_L // 8 // _LB,),
    in_specs=[pl.BlockSpec((_LB, K), lambda i: (i, 0)), _full_spec(K, 8)],
    out_specs=pl.BlockSpec((_LB, 8), lambda i: (i, 0)),
    out_shape=jax.ShapeDtypeStruct((EPAD_L // 8, 8), jnp.float32),
)


def _tile_edges(idx, nway, nchunk):
    pad = nway * nchunk * K - idx.shape[0]
    idx = jnp.concatenate([idx, jnp.full((pad,), N, dtype=jnp.int32)])
    return idx.reshape(nway, nchunk, K)


def _blk4(t):
    return t.reshape(t.shape[0], -1, NBI, K)


def kernel(x_gene, x_cell, W1_gg, b1_gg, W1_rev, b1_rev, W1_cc, b1_cc,
           W2_gg, b2_gg, W2_rev, b2_rev, W2_cc, b2_cc,
           edge_index_gg, edge_index_gg_rev, edge_index_cc, edge_label_index):
    x = jnp.pad(x_gene, ((0, NPAD - N), (0, 0)))
    src_gg = _tile_edges(edge_index_gg[0], NS, NCHUNK)
    dst_gg = _tile_edges(edge_index_gg[1], NS, NCHUNK)
    src_rev = _tile_edges(edge_index_gg_rev[0], NS, NCHUNK)
    dst_rev = _tile_edges(edge_index_gg_rev[1], NS, NCHUNK)
    src_gg2 = _tile_edges(edge_index_gg[0], NS * NC, NCHUNK2)
    dst_gg2 = _tile_edges(edge_index_gg[1], NS * NC, NCHUNK2)
    src_rev2 = _tile_edges(edge_index_gg_rev[0], NS * NC, NCHUNK2)
    dst_rev2 = _tile_edges(edge_index_gg_rev[1], NS * NC, NCHUNK2)
    l0 = _tile_edges(edge_label_index[0], NS * NC, NCHUNK_L)
    l1 = _tile_edges(edge_label_index[1], NS * NC, NCHUNK_L)

    zeros1 = jnp.zeros((NPAD,), jnp.float32)
    z128 = jnp.zeros((NPAD, H1 // 2), jnp.float32)

    dst_cat = jnp.concatenate([dst_gg, dst_rev], axis=0).reshape(NC * NS, -1)
    hist = _degrees(dst_cat, zeros1)
    dinv_gg, dinv_rev = _tc0(hist[0], hist[1])
    dinv_gg = dinv_gg.reshape(NPAD, 1)
    dinv_rev = dinv_rev.reshape(NPAD, 1)

    ygg_lo, ygg_hi, yrev_lo, yrev_hi = _tc1(x, W1_gg, W1_rev, dinv_gg, dinv_rev)

    src_gg_cat = jnp.concatenate([src_gg, src_gg + NPAD], axis=0)
    src_rev_cat = jnp.concatenate([src_rev, src_rev + NPAD], axis=0)
    ygg_cat = jnp.concatenate([ygg_lo, ygg_hi], axis=0)
    yrev_cat = jnp.concatenate([yrev_lo, yrev_hi], axis=0)
    agg_gg = _scatter_h1(ygg_cat, _blk4(src_gg_cat), _blk4(dst_gg), z128)
    agg_rev = _scatter_h1(yrev_cat, _blk4(src_rev_cat), _blk4(dst_rev), z128)

    y2gg, y2rev = _tc2(
        agg_gg[0], agg_gg[1], agg_rev[0], agg_rev[1],
        ygg_lo, ygg_hi, yrev_lo, yrev_hi, dinv_gg, dinv_rev,
        b1_gg.reshape(1, H1), b1_rev.reshape(1, H1), W2_gg, W2_rev)

    agg2_gg = _scatter_h2(y2gg, _blk4(src_gg2), _blk4(dst_gg2), z128)
    agg2_rev = _scatter_h2(y2rev, _blk4(src_rev2), _blk4(dst_rev2), z128)

    g2 = _tc3(agg2_gg[0], agg2_gg[1], agg2_rev[0], agg2_rev[1],
              y2gg, y2rev, dinv_gg, dinv_rev,
              b2_gg.reshape(1, H2), b2_rev.reshape(1, H2))

    pred16 = _label_dot(g2, l0, l1)
    sel = (jnp.arange(K)[:, None] // 16 == jnp.arange(8)[None, :]).astype(jnp.float32)
    pred = _tc4(pred16, sel)
    return pred.reshape(EPAD_L)[:E_LBL]


# speedup vs baseline: 6.5236x; 1.0021x over previous
import functools

import jax
import jax.numpy as jnp
from jax import lax
from jax.experimental import pallas as pl
from jax.experimental.pallas import tpu as pltpu
from jax.experimental.pallas import tpu_sc as plsc

N = 10000
NPAD = 10240
D_IN = 128
H1 = 256
H2 = 128
E = 320000
E_LBL = 100000

NS = 16
NC = 2
K = 128
NCHUNK = 160
NCHUNK2 = 80
NCHUNK_L = 26
NBI = 8
EPAD_L = NS * NC * K * NCHUNK_L
ROWS_PER_TILE = NPAD // NS
EDGES_PER_TILE = NCHUNK * K

_MESH = plsc.VectorSubcoreMesh(core_axis_name="c", subcore_axis_name="s")


def _deg_body(dst_cat, zeros8, hist, idx_v, acc_t, sem):
    c = lax.axis_index("c")
    s = lax.axis_index("s")
    w = c * NS + s
    pltpu.sync_copy(zeros8, acc_t)
    pltpu.sync_copy(dst_cat.at[w], idx_v)
    ones = jnp.ones((16,), jnp.float32)

    def grp(m, carry):
        base = m * K
        for u in range(8):
            iv = idx_v[pl.ds(base + u * 16, 16)]
            uv = jnp.full((16,), u, dtype=jnp.int32)
            plsc.addupdate_scatter(acc_t, [uv, iv], ones)
        return carry

    lax.fori_loop(0, EDGES_PER_TILE // K, grp, 0)
    pltpu.sync_copy(acc_t, hist.at[w])


_degrees = pl.kernel(
    _deg_body,
    mesh=_MESH,
    compiler_params=pltpu.CompilerParams(needs_layout_passes=False),
    out_type=jax.ShapeDtypeStruct((NC * NS, 8, NPAD), jnp.float32),
    scratch_types=[pltpu.VMEM((EDGES_PER_TILE,), jnp.int32),
                   pltpu.VMEM((8, NPAD), jnp.float32),
                   pltpu.SemaphoreType.DMA],
)


def _pipelined_scatter(y, src_t, dst_t, acc, w_src, w_dst, nchunk,
                       isa, ida, isb, idb, rows_a, rows_b,
                       ga, sa, gb, sb, ia, ib):
    def astage(b, i_s, i_d, sem):
        pltpu.async_copy(src_t.at[w_src, b], i_s, sem)
        pltpu.async_copy(dst_t.at[w_dst, b], i_d, sem)

    def wstage(i_s, i_d, sem):
        pltpu.make_async_copy(src_t.at[w_src, 0], i_s, sem).wait()
        pltpu.make_async_copy(dst_t.at[w_dst, 0], i_d, sem).wait()

    def inner(i_s, i_d):
        pltpu.async_copy(y.at[i_s.at[0]], rows_a, ga)
        for jj in range(NBI // 2):
            j0 = 2 * jj
            if jj > 0:
                pltpu.make_async_copy(rows_b, acc.at[i_d.at[j0 - 1]], sb).wait()
            pltpu.async_copy(y.at[i_s.at[j0 + 1]], rows_b, gb)
            pltpu.make_async_copy(y.at[i_s.at[j0]], rows_a, ga).wait()
            pltpu.async_copy(rows_a, acc.at[i_d.at[j0]], sa, add=True)
            if jj < NBI // 2 - 1:
                pltpu.make_async_copy(rows_a, acc.at[i_d.at[j0]], sa).wait()
                pltpu.async_copy(y.at[i_s.at[j0 + 2]], rows_a, ga)
            pltpu.make_async_copy(y.at[i_s.at[j0 + 1]], rows_b, gb).wait()
            pltpu.async_copy(rows_b, acc.at[i_d.at[j0 + 1]], sb, add=True)
        pltpu.make_async_copy(rows_a, acc.at[i_d.at[NBI - 2]], sa).wait()
        pltpu.make_async_copy(rows_b, acc.at[i_d.at[NBI - 1]], sb).wait()

    nb2 = nchunk // NBI // 2
    astage(0, isa, ida, ia)
    astage(1, isb, idb, ib)

    def outer(bb, carry):
        wstage(isa, ida, ia)
        inner(isa, ida)

        @pl.when(bb + 1 < nb2)
        def _():
            astage(2 * bb + 2, isa, ida, ia)

        wstage(isb, idb, ib)
        inner(isb, idb)

        @pl.when(bb + 1 < nb2)
        def _():
            astage(2 * bb + 3, isb, idb, ib)

        return carry

    lax.fori_loop(0, nb2, outer, 0)


def _scatter_scratch(dh):
    return [pltpu.VMEM((NBI, K), jnp.int32),
            pltpu.VMEM((NBI, K), jnp.int32),
            pltpu.VMEM((NBI, K), jnp.int32),
            pltpu.VMEM((NBI, K), jnp.int32),
            pltpu.VMEM((K, dh), jnp.float32),
            pltpu.VMEM((K, dh), jnp.float32),
            pltpu.VMEM_SHARED((NPAD, dh), jnp.float32),
            pltpu.SemaphoreType.DMA,
            pltpu.SemaphoreType.DMA,
            pltpu.SemaphoreType.DMA,
            pltpu.SemaphoreType.DMA,
            pltpu.SemaphoreType.DMA,
            pltpu.SemaphoreType.DMA]


def _scatter1_body(y_cat, src_t, dst_t, zeros, agg,
                   isa, ida, isb, idb, rows_a, rows_b,
                   acc, ga, sa, gb, sb, ia, ib):
    c = lax.axis_index("c")
    s = lax.axis_index("s")
    w = c * NS + s
    rows = pl.ds(s * ROWS_PER_TILE, ROWS_PER_TILE)
    pltpu.sync_copy(zeros.at[rows], acc.at[rows])
    plsc.subcore_barrier()
    _pipelined_scatter(y_cat, src_t, dst_t, acc, w, s, NCHUNK,
                       isa, ida, isb, idb, rows_a, rows_b,
                       ga, sa, gb, sb, ia, ib)
    plsc.subcore_barrier()
    pltpu.sync_copy(acc.at[rows], agg.at[c, rows])


_scatter_h1 = pl.kernel(
    _scatter1_body,
    mesh=_MESH,
    out_type=jax.ShapeDtypeStruct((NC, NPAD, H1 // 2), jnp.float32),
    scratch_types=_scatter_scratch(H1 // 2),
)


def _scatter2_body(y2, src_t, dst_t, zeros, part,
                   isa, ida, isb, idb, rows_a, rows_b,
                   acc, ga, sa, gb, sb, ia, ib):
    c = lax.axis_index("c")
    s = lax.axis_index("s")
    w = s * NC + c
    rows = pl.ds(s * ROWS_PER_TILE, ROWS_PER_TILE)
    pltpu.sync_copy(zeros.at[rows], acc.at[rows])
    plsc.subcore_barrier()
    _pipelined_scatter(y2, src_t, dst_t, acc, w, w, NCHUNK2,
                       isa, ida, isb, idb, rows_a, rows_b,
                       ga, sa, gb, sb, ia, ib)
    plsc.subcore_barrier()
    pltpu.sync_copy(acc.at[rows], part.at[c, rows])


_scatter_h2 = pl.kernel(
    _scatter2_body,
    mesh=_MESH,
    out_type=jax.ShapeDtypeStruct((NC, NPAD, H2), jnp.float32),
    scratch_types=_scatter_scratch(H2),
)


LBL_PER_TILE = NCHUNK_L * K
LBL_ROWS = LBL_PER_TILE // 8


def _label_body(g2, l0_t, l1_t, pred, i0a, i1a, r0a, r1a, i0b, i1b, r0b, r1b,
                out_v, s0a, s1a, s0b, s1b):
    c = lax.axis_index("c")
    s = lax.axis_index("s")
    w = s * NC + c

    def stage(j, i0_v, i1_v, r0_v, r1_v, sg0, sg1):
        pltpu.sync_copy(l0_t.at[w, j], i0_v)
        pltpu.sync_copy(l1_t.at[w, j], i1_v)
        pltpu.async_copy(g2.at[i0_v], r0_v, sg0)
        pltpu.async_copy(g2.at[i1_v], r1_v, sg1)

    def compute(j, i0_v, i1_v, r0_v, r1_v, sg0, sg1):
        pltpu.make_async_copy(g2.at[i0_v], r0_v, sg0).wait()
        pltpu.make_async_copy(g2.at[i1_v], r1_v, sg1).wait()

        def edge(e, carry2):
            acc = r0_v[e, pl.ds(0, 16)] * r1_v[e, pl.ds(0, 16)]
            for k in range(1, H2 // 16):
                acc = acc + r0_v[e, pl.ds(k * 16, 16)] * r1_v[e, pl.ds(k * 16, 16)]
            out_v[j * (K // 8) + e // 8, pl.ds((e % 8) * 16, 16)] = acc
            return carry2

        lax.fori_loop(0, K, edge, 0)

    stage(0, i0a, i1a, r0a, r1a, s0a, s1a)

    def body(jj, carry):
        j0 = 2 * jj
        stage(j0 + 1, i0b, i1b, r0b, r1b, s0b, s1b)
        compute(j0, i0a, i1a, r0a, r1a, s0a, s1a)

        @pl.when(jj + 1 < NCHUNK_L // 2)
        def _():
            stage(j0 + 2, i0a, i1a, r0a, r1a, s0a, s1a)

        compute(j0 + 1, i0b, i1b, r0b, r1b, s0b, s1b)
        return carry

    lax.fori_loop(0, NCHUNK_L // 2, body, 0)
    pltpu.sync_copy(out_v, pred.at[pl.ds(w * LBL_ROWS, LBL_ROWS)])


_label_dot = pl.kernel(
    _label_body,
    mesh=_MESH,
    out_type=jax.ShapeDtypeStruct((EPAD_L // 8, K), jnp.float32),
    scratch_types=[pltpu.VMEM((K,), jnp.int32),
                   pltpu.VMEM((K,), jnp.int32),
                   pltpu.VMEM((K, H2), jnp.float32),
                   pltpu.VMEM((K, H2), jnp.float32),
                   pltpu.VMEM((K,), jnp.int32),
                   pltpu.VMEM((K,), jnp.int32),
                   pltpu.VMEM((K, H2), jnp.float32),
                   pltpu.VMEM((K, H2), jnp.float32),
                   pltpu.VMEM((LBL_ROWS, K), jnp.float32),
                   pltpu.SemaphoreType.DMA,
                   pltpu.SemaphoreType.DMA,
                   pltpu.SemaphoreType.DMA,
                   pltpu.SemaphoreType.DMA],
)


_RB = 512
_GRID = NPAD // _RB
_CB = 1024


def _tc0_body(hgg, hrev, dgg, drev):
    dgg[...] = lax.rsqrt(jnp.sum(hgg[...], axis=0, keepdims=True) + 1.0)
    drev[...] = lax.rsqrt(jnp.sum(hrev[...], axis=0, keepdims=True) + 1.0)


_tc0 = pl.pallas_call(
    _tc0_body,
    out_shape=[jax.ShapeDtypeStruct((1, NPAD), jnp.float32)] * 2,
)


def _tc1_body(x, w_gg, w_rev, dgg, drev, ygg_lo, ygg_hi, yrev_lo, yrev_hi):
    y = jnp.dot(x[...], w_gg[...], preferred_element_type=jnp.float32) * dgg[...]
    ygg_lo[...] = y[:, :H1 // 2]
    ygg_hi[...] = y[:, H1 // 2:]
    y = jnp.dot(x[...], w_rev[...], preferred_element_type=jnp.float32) * drev[...]
    yrev_lo[...] = y[:, :H1 // 2]
    yrev_hi[...] = y[:, H1 // 2:]


def _row_spec(w):
    return pl.BlockSpec((_RB, w), lambda i: (i, 0))


def _full_spec(h, w):
    return pl.BlockSpec((h, w), lambda i: (0, 0))


_tc1 = pl.pallas_call(
    _tc1_body,
    grid=(_GRID,),
    in_specs=[_row_spec(D_IN), _full_spec(D_IN, H1), _full_spec(D_IN, H1),
              _row_spec(1), _row_spec(1)],
    out_specs=[_row_spec(H1 // 2)] * 4,
    out_shape=[jax.ShapeDtypeStruct((NPAD, H1 // 2), jnp.float32)] * 4,
)


def _tc2_body(agg_gg_lo, agg_gg_hi, agg_rev_lo, agg_rev_hi,
              ygg_lo, ygg_hi, yrev_lo, yrev_hi, dgg, drev,
              b_gg, b_rev, w2_gg, w2_rev, y2gg, y2rev):
    agg_gg = jnp.concatenate([agg_gg_lo[...] + ygg_lo[...],
                              agg_gg_hi[...] + ygg_hi[...]], axis=1)
    agg_rev = jnp.concatenate([agg_rev_lo[...] + yrev_lo[...],
                               agg_rev_hi[...] + yrev_hi[...]], axis=1)
    di_gg = dgg[...]
    di_rev = drev[...]
    g = jax.nn.relu(di_gg * agg_gg + b_gg[...] + di_rev * agg_rev + b_rev[...])
    y2gg[...] = jnp.dot(g, w2_gg[...], preferred_element_type=jnp.float32) * di_gg
    y2rev[...] = jnp.dot(g, w2_rev[...], preferred_element_type=jnp.float32) * di_rev


_tc2 = pl.pallas_call(
    _tc2_body,
    grid=(_GRID,),
    in_specs=[_row_spec(H1 // 2)] * 8 + [_row_spec(1)] * 2
             + [_full_spec(1, H1)] * 2 + [_full_spec(H1, H2)] * 2,
    out_specs=[_row_spec(H2)] * 2,
    out_shape=[jax.ShapeDtypeStruct((NPAD, H2), jnp.float32)] * 2,
)


def _tc3_body(agg_gg_p0, agg_gg_p1, agg_rev_p0, agg_rev_p1,
              y2gg, y2rev, dgg, drev, b_gg, b_rev, g2):
    a_gg = agg_gg_p0[...] + agg_gg_p1[...] + y2gg[...]
    a_rev = agg_rev_p0[...] + agg_rev_p1[...] + y2rev[...]
    g2[...] = (dgg[...] * a_gg + b_gg[...] + drev[...] * a_rev + b_rev[...])


_tc3 = pl.pallas_call(
    _tc3_body,
    grid=(_GRID,),
    in_specs=[_row_spec(H2)] * 6 + [_row_spec(1)] * 2
             + [_full_spec(1, H2)] * 2,
    out_specs=_row_spec(H2),
    out_shape=jax.ShapeDtypeStruct((NPAD, H2), jnp.float32),
)


def _tc4_body(p16, sel, pred):
    pred[...] = jnp.dot(p16[...], sel[...], preferred_element_type=jnp.float32)


_LB = 3328

_tc4 = pl.pallas_call(
    _tc4_body,
    grid=(EPAD_L // 8 // _LB,),
    in_specs=[pl.BlockSpec((_LB, K), lambda i: (i, 0)), _full_spec(K, 8)],
    out_specs=pl.BlockSpec((_LB, 8), lambda i: (i, 0)),
    out_shape=jax.ShapeDtypeStruct((EPAD_L // 8, 8), jnp.float32),
)


def _tile_edges(idx, nway, nchunk):
    pad = nway * nchunk * K - idx.shape[0]
    idx = jnp.concatenate([idx, jnp.full((pad,), N, dtype=jnp.int32)])
    return idx.reshape(nway, nchunk, K)


def _blk4(t):
    return t.reshape(t.shape[0], -1, NBI, K)


def kernel(x_gene, x_cell, W1_gg, b1_gg, W1_rev, b1_rev, W1_cc, b1_cc,
           W2_gg, b2_gg, W2_rev, b2_rev, W2_cc, b2_cc,
           edge_index_gg, edge_index_gg_rev, edge_index_cc, edge_label_index):
    x = jnp.pad(x_gene, ((0, NPAD - N), (0, 0)))
    src_gg = _tile_edges(edge_index_gg[0], NS, NCHUNK)
    dst_gg = _tile_edges(edge_index_gg[1], NS, NCHUNK)
    src_rev = _tile_edges(edge_index_gg_rev[0], NS, NCHUNK)
    dst_rev = _tile_edges(edge_index_gg_rev[1], NS, NCHUNK)
    src_gg2 = _tile_edges(edge_index_gg[0], NS * NC, NCHUNK2)
    dst_gg2 = _tile_edges(edge_index_gg[1], NS * NC, NCHUNK2)
    src_rev2 = _tile_edges(edge_index_gg_rev[0], NS * NC, NCHUNK2)
    dst_rev2 = _tile_edges(edge_index_gg_rev[1], NS * NC, NCHUNK2)
    l0 = _tile_edges(edge_label_index[0], NS * NC, NCHUNK_L)
    l1 = _tile_edges(edge_label_index[1], NS * NC, NCHUNK_L)

    zeros8 = jnp.zeros((8, NPAD), jnp.float32)
    z128 = jnp.zeros((NPAD, H1 // 2), jnp.float32)

    dst_cat = jnp.concatenate([dst_gg, dst_rev], axis=0).reshape(NC * NS, -1)
    hist = _degrees(dst_cat, zeros8).reshape(NC, NS * 8, NPAD)
    dinv_gg, dinv_rev = _tc0(hist[0], hist[1])
    dinv_gg = dinv_gg.reshape(NPAD, 1)
    dinv_rev = dinv_rev.reshape(NPAD, 1)

    ygg_lo, ygg_hi, yrev_lo, yrev_hi = _tc1(x, W1_gg, W1_rev, dinv_gg, dinv_rev)

    src_gg_cat = jnp.concatenate([src_gg, src_gg + NPAD], axis=0)
    src_rev_cat = jnp.concatenate([src_rev, src_rev + NPAD], axis=0)
    ygg_cat = jnp.concatenate([ygg_lo, ygg_hi], axis=0)
    yrev_cat = jnp.concatenate([yrev_lo, yrev_hi], axis=0)
    agg_gg = _scatter_h1(ygg_cat, _blk4(src_gg_cat), _blk4(dst_gg), z128)
    agg_rev = _scatter_h1(yrev_cat, _blk4(src_rev_cat), _blk4(dst_rev), z128)

    y2gg, y2rev = _tc2(
        agg_gg[0], agg_gg[1], agg_rev[0], agg_rev[1],
        ygg_lo, ygg_hi, yrev_lo, yrev_hi, dinv_gg, dinv_rev,
        b1_gg.reshape(1, H1), b1_rev.reshape(1, H1), W2_gg, W2_rev)

    agg2_gg = _scatter_h2(y2gg, _blk4(src_gg2), _blk4(dst_gg2), z128)
    agg2_rev = _scatter_h2(y2rev, _blk4(src_rev2), _blk4(dst_rev2), z128)

    g2 = _tc3(agg2_gg[0], agg2_gg[1], agg2_rev[0], agg2_rev[1],
              y2gg, y2rev, dinv_gg, dinv_rev,
              b2_gg.reshape(1, H2), b2_rev.reshape(1, H2))

    pred16 = _label_dot(g2, l0, l1)
    sel = (jnp.arange(K)[:, None] // 16 == jnp.arange(8)[None, :]).astype(jnp.float32)
    pred = _tc4(pred16, sel)
    return pred.reshape(EPAD_L)[:E_LBL]


# speedup vs baseline: 6.6476x; 1.0190x over previous
import functools

import jax
import jax.numpy as jnp
from jax import lax
from jax.experimental import pallas as pl
from jax.experimental.pallas import tpu as pltpu
from jax.experimental.pallas import tpu_sc as plsc

N = 10000
NPAD = 10240
D_IN = 128
H1 = 256
H2 = 128
E = 320000
E_LBL = 100000

NS = 16
NC = 2
K = 128
NCHUNK = 160
NCHUNK2 = 80
NCHUNK_L = 26
NBI = 8
EPAD_L = NS * NC * K * NCHUNK_L
ROWS_PER_TILE = NPAD // NS
EDGES_PER_TILE = NCHUNK * K

_MESH = plsc.VectorSubcoreMesh(core_axis_name="c", subcore_axis_name="s")


def _deg_body(dst_cat, zeros8, hist, idx_v, acc_t, sem):
    c = lax.axis_index("c")
    s = lax.axis_index("s")
    w = c * NS + s
    pltpu.sync_copy(zeros8, acc_t)
    pltpu.sync_copy(dst_cat.at[w], idx_v)
    ones = jnp.ones((16,), jnp.float32)

    def grp(m, carry):
        base = m * K
        for u in range(8):
            iv = idx_v[pl.ds(base + u * 16, 16)]
            uv = jnp.full((16,), u, dtype=jnp.int32)
            plsc.addupdate_scatter(acc_t, [uv, iv], ones)
        return carry

    lax.fori_loop(0, EDGES_PER_TILE // K, grp, 0)
    pltpu.sync_copy(acc_t, hist.at[w])


_degrees = pl.kernel(
    _deg_body,
    mesh=_MESH,
    compiler_params=pltpu.CompilerParams(needs_layout_passes=False),
    out_type=jax.ShapeDtypeStruct((NC * NS, 8, NPAD), jnp.float32),
    scratch_types=[pltpu.VMEM((EDGES_PER_TILE,), jnp.int32),
                   pltpu.VMEM((8, NPAD), jnp.float32),
                   pltpu.SemaphoreType.DMA],
)


def _pipelined_scatter(y, src_t, dst_t, acc, w_src, w_dst, nchunk,
                       isa, ida, isb, idb, rows_a, rows_b,
                       ga, sa, gb, sb, ia, ib):
    def astage(b, i_s, i_d, sem):
        pltpu.async_copy(src_t.at[w_src, b], i_s, sem)
        pltpu.async_copy(dst_t.at[w_dst, b], i_d, sem)

    def wstage(i_s, i_d, sem):
        pltpu.make_async_copy(src_t.at[w_src, 0], i_s, sem).wait()
        pltpu.make_async_copy(dst_t.at[w_dst, 0], i_d, sem).wait()

    def inner(i_s, i_d):
        pltpu.async_copy(y.at[i_s.at[0]], rows_a, ga)
        for jj in range(NBI // 2):
            j0 = 2 * jj
            if jj > 0:
                pltpu.make_async_copy(rows_b, acc.at[i_d.at[j0 - 1]], sb).wait()
            pltpu.async_copy(y.at[i_s.at[j0 + 1]], rows_b, gb)
            pltpu.make_async_copy(y.at[i_s.at[j0]], rows_a, ga).wait()
            pltpu.async_copy(rows_a, acc.at[i_d.at[j0]], sa, add=True)
            if jj < NBI // 2 - 1:
                pltpu.make_async_copy(rows_a, acc.at[i_d.at[j0]], sa).wait()
                pltpu.async_copy(y.at[i_s.at[j0 + 2]], rows_a, ga)
            pltpu.make_async_copy(y.at[i_s.at[j0 + 1]], rows_b, gb).wait()
            pltpu.async_copy(rows_b, acc.at[i_d.at[j0 + 1]], sb, add=True)
        pltpu.make_async_copy(rows_a, acc.at[i_d.at[NBI - 2]], sa).wait()
        pltpu.make_async_copy(rows_b, acc.at[i_d.at[NBI - 1]], sb).wait()

    nb2 = nchunk // NBI // 2
    astage(0, isa, ida, ia)
    astage(1, isb, idb, ib)

    def outer(bb, carry):
        wstage(isa, ida, ia)
        inner(isa, ida)

        @pl.when(bb + 1 < nb2)
        def _():
            astage(2 * bb + 2, isa, ida, ia)

        wstage(isb, idb, ib)
        inner(isb, idb)

        @pl.when(bb + 1 < nb2)
        def _():
            astage(2 * bb + 3, isb, idb, ib)

        return carry

    lax.fori_loop(0, nb2, outer, 0)


def _scatter_scratch(dh):
    return [pltpu.VMEM((NBI, K), jnp.int32),
            pltpu.VMEM((NBI, K), jnp.int32),
            pltpu.VMEM((NBI, K), jnp.int32),
            pltpu.VMEM((NBI, K), jnp.int32),
            pltpu.VMEM((K, dh), jnp.float32),
            pltpu.VMEM((K, dh), jnp.float32),
            pltpu.VMEM_SHARED((NPAD, dh), jnp.float32),
            pltpu.SemaphoreType.DMA,
            pltpu.SemaphoreType.DMA,
            pltpu.SemaphoreType.DMA,
            pltpu.SemaphoreType.DMA,
            pltpu.SemaphoreType.DMA,
            pltpu.SemaphoreType.DMA]


def _scatter1_body(ygg_cat, yrev_cat, sgg_t, srev_t, dgg_t, drev_t, zeros, agg,
                   isa, ida, isb, idb, rows_a, rows_b,
                   acc, ga, sa, gb, sb, ia, ib):
    c = lax.axis_index("c")
    s = lax.axis_index("s")
    w = c * NS + s
    rows = pl.ds(s * ROWS_PER_TILE, ROWS_PER_TILE)
    for r, (y, s_t, d_t) in enumerate([(ygg_cat, sgg_t, dgg_t),
                                       (yrev_cat, srev_t, drev_t)]):
        pltpu.sync_copy(zeros.at[rows], acc.at[rows])
        plsc.subcore_barrier()
        _pipelined_scatter(y, s_t, d_t, acc, w, s, NCHUNK,
                           isa, ida, isb, idb, rows_a, rows_b,
                           ga, sa, gb, sb, ia, ib)
        plsc.subcore_barrier()
        pltpu.sync_copy(acc.at[rows], agg.at[r, c, rows])
        plsc.subcore_barrier()


_scatter_h1 = pl.kernel(
    _scatter1_body,
    mesh=_MESH,
    out_type=jax.ShapeDtypeStruct((2, NC, NPAD, H1 // 2), jnp.float32),
    scratch_types=_scatter_scratch(H1 // 2),
)


def _scatter2_body(y2gg, y2rev, sgg_t, srev_t, dgg_t, drev_t, zeros, part,
                   isa, ida, isb, idb, rows_a, rows_b,
                   acc, ga, sa, gb, sb, ia, ib):
    c = lax.axis_index("c")
    s = lax.axis_index("s")
    w = s * NC + c
    rows = pl.ds(s * ROWS_PER_TILE, ROWS_PER_TILE)
    for r, (y, s_t, d_t) in enumerate([(y2gg, sgg_t, dgg_t),
                                       (y2rev, srev_t, drev_t)]):
        pltpu.sync_copy(zeros.at[rows], acc.at[rows])
        plsc.subcore_barrier()
        _pipelined_scatter(y, s_t, d_t, acc, w, w, NCHUNK2,
                           isa, ida, isb, idb, rows_a, rows_b,
                           ga, sa, gb, sb, ia, ib)
        plsc.subcore_barrier()
        pltpu.sync_copy(acc.at[rows], part.at[r, c, rows])
        plsc.subcore_barrier()


_scatter_h2 = pl.kernel(
    _scatter2_body,
    mesh=_MESH,
    out_type=jax.ShapeDtypeStruct((2, NC, NPAD, H2), jnp.float32),
    scratch_types=_scatter_scratch(H2),
)


LBL_PER_TILE = NCHUNK_L * K
LBL_ROWS = LBL_PER_TILE // 8


def _label_body(g2, l0_t, l1_t, pred, i0a, i1a, r0a, r1a, i0b, i1b, r0b, r1b,
                out_v, s0a, s1a, s0b, s1b):
    c = lax.axis_index("c")
    s = lax.axis_index("s")
    w = s * NC + c

    def stage(j, i0_v, i1_v, r0_v, r1_v, sg0, sg1):
        pltpu.sync_copy(l0_t.at[w, j], i0_v)
        pltpu.sync_copy(l1_t.at[w, j], i1_v)
        pltpu.async_copy(g2.at[i0_v], r0_v, sg0)
        pltpu.async_copy(g2.at[i1_v], r1_v, sg1)

    def compute(j, i0_v, i1_v, r0_v, r1_v, sg0, sg1):
        pltpu.make_async_copy(g2.at[i0_v], r0_v, sg0).wait()
        pltpu.make_async_copy(g2.at[i1_v], r1_v, sg1).wait()

        def edge(ee, carry2):
            for u in range(4):
                e = ee * 4 + u
                acc = r0_v[e, pl.ds(0, 16)] * r1_v[e, pl.ds(0, 16)]
                for k in range(1, H2 // 16):
                    acc = acc + r0_v[e, pl.ds(k * 16, 16)] * r1_v[e, pl.ds(k * 16, 16)]
                out_v[j * (K // 8) + e // 8, pl.ds((e % 8) * 16, 16)] = acc
            return carry2

        lax.fori_loop(0, K // 4, edge, 0)

    stage(0, i0a, i1a, r0a, r1a, s0a, s1a)

    def body(jj, carry):
        j0 = 2 * jj
        stage(j0 + 1, i0b, i1b, r0b, r1b, s0b, s1b)
        compute(j0, i0a, i1a, r0a, r1a, s0a, s1a)

        @pl.when(jj + 1 < NCHUNK_L // 2)
        def _():
            stage(j0 + 2, i0a, i1a, r0a, r1a, s0a, s1a)

        compute(j0 + 1, i0b, i1b, r0b, r1b, s0b, s1b)
        return carry

    lax.fori_loop(0, NCHUNK_L // 2, body, 0)
    pltpu.sync_copy(out_v, pred.at[pl.ds(w * LBL_ROWS, LBL_ROWS)])


_label_dot = pl.kernel(
    _label_body,
    mesh=_MESH,
    out_type=jax.ShapeDtypeStruct((EPAD_L // 8, K), jnp.float32),
    scratch_types=[pltpu.VMEM((K,), jnp.int32),
                   pltpu.VMEM((K,), jnp.int32),
                   pltpu.VMEM((K, H2), jnp.float32),
                   pltpu.VMEM((K, H2), jnp.float32),
                   pltpu.VMEM((K,), jnp.int32),
                   pltpu.VMEM((K,), jnp.int32),
                   pltpu.VMEM((K, H2), jnp.float32),
                   pltpu.VMEM((K, H2), jnp.float32),
                   pltpu.VMEM((LBL_ROWS, K), jnp.float32),
                   pltpu.SemaphoreType.DMA,
                   pltpu.SemaphoreType.DMA,
                   pltpu.SemaphoreType.DMA,
                   pltpu.SemaphoreType.DMA],
)


_RB = 512
_GRID = NPAD // _RB
_CB = 1024


def _tc0_body(hgg, hrev, dgg, drev):
    dgg[...] = lax.rsqrt(jnp.sum(hgg[...], axis=0, keepdims=True) + 1.0)
    drev[...] = lax.rsqrt(jnp.sum(hrev[...], axis=0, keepdims=True) + 1.0)


_tc0 = pl.pallas_call(
    _tc0_body,
    out_shape=[jax.ShapeDtypeStruct((1, NPAD), jnp.float32)] * 2,
)


def _tc1_body(x, w_gg, w_rev, dgg, drev, ygg_lo, ygg_hi, yrev_lo, yrev_hi):
    y = jnp.dot(x[...], w_gg[...], preferred_element_type=jnp.float32) * dgg[...]
    ygg_lo[...] = y[:, :H1 // 2]
    ygg_hi[...] = y[:, H1 // 2:]
    y = jnp.dot(x[...], w_rev[...], preferred_element_type=jnp.float32) * drev[...]
    yrev_lo[...] = y[:, :H1 // 2]
    yrev_hi[...] = y[:, H1 // 2:]


def _row_spec(w):
    return pl.BlockSpec((_RB, w), lambda i: (i, 0))


def _full_spec(h, w):
    return pl.BlockSpec((h, w), lambda i: (0, 0))


_tc1 = pl.pallas_call(
    _tc1_body,
    grid=(_GRID,),
    in_specs=[_row_spec(D_IN), _full_spec(D_IN, H1), _full_spec(D_IN, H1),
              _row_spec(1), _row_spec(1)],
    out_specs=[_row_spec(H1 // 2)] * 4,
    out_shape=[jax.ShapeDtypeStruct((NPAD, H1 // 2), jnp.float32)] * 4,
)


def _tc2_body(agg_gg_lo, agg_gg_hi, agg_rev_lo, agg_rev_hi,
              ygg_lo, ygg_hi, yrev_lo, yrev_hi, dgg, drev,
              b_gg, b_rev, w2_gg, w2_rev, y2gg, y2rev):
    agg_gg = jnp.concatenate([agg_gg_lo[...] + ygg_lo[...],
                              agg_gg_hi[...] + ygg_hi[...]], axis=1)
    agg_rev = jnp.concatenate([agg_rev_lo[...] + yrev_lo[...],
                               agg_rev_hi[...] + yrev_hi[...]], axis=1)
    di_gg = dgg[...]
    di_rev = drev[...]
    g = jax.nn.relu(di_gg * agg_gg + b_gg[...] + di_rev * agg_rev + b_rev[...])
    y2gg[...] = jnp.dot(g, w2_gg[...], preferred_element_type=jnp.float32) * di_gg
    y2rev[...] = jnp.dot(g, w2_rev[...], preferred_element_type=jnp.float32) * di_rev


_tc2 = pl.pallas_call(
    _tc2_body,
    grid=(_GRID,),
    in_specs=[_row_spec(H1 // 2)] * 8 + [_row_spec(1)] * 2
             + [_full_spec(1, H1)] * 2 + [_full_spec(H1, H2)] * 2,
    out_specs=[_row_spec(H2)] * 2,
    out_shape=[jax.ShapeDtypeStruct((NPAD, H2), jnp.float32)] * 2,
)


def _tc3_body(agg_gg_p0, agg_gg_p1, agg_rev_p0, agg_rev_p1,
              y2gg, y2rev, dgg, drev, b_gg, b_rev, g2):
    a_gg = agg_gg_p0[...] + agg_gg_p1[...] + y2gg[...]
    a_rev = agg_rev_p0[...] + agg_rev_p1[...] + y2rev[...]
    g2[...] = (dgg[...] * a_gg + b_gg[...] + drev[...] * a_rev + b_rev[...])


_tc3 = pl.pallas_call(
    _tc3_body,
    grid=(_GRID,),
    in_specs=[_row_spec(H2)] * 6 + [_row_spec(1)] * 2
             + [_full_spec(1, H2)] * 2,
    out_specs=_row_spec(H2),
    out_shape=jax.ShapeDtypeStruct((NPAD, H2), jnp.float32),
)


def _tc4_body(p16, sel, pred):
    pred[...] = jnp.dot(p16[...], sel[...], preferred_element_type=jnp.float32)


_LB = 3328

_tc4 = pl.pallas_call(
    _tc4_body,
    grid=(EPAD_L // 8 // _LB,),
    in_specs=[pl.BlockSpec((_LB, K), lambda i: (i, 0)), _full_spec(K, 8)],
    out_specs=pl.BlockSpec((_LB, 8), lambda i: (i, 0)),
    out_shape=jax.ShapeDtypeStruct((EPAD_L // 8, 8), jnp.float32),
)


def _tile_edges(idx, nway, nchunk):
    pad = nway * nchunk * K - idx.shape[0]
    idx = jnp.concatenate([idx, jnp.full((pad,), N, dtype=jnp.int32)])
    return idx.reshape(nway, nchunk, K)


def _blk4(t):
    return t.reshape(t.shape[0], -1, NBI, K)


def kernel(x_gene, x_cell, W1_gg, b1_gg, W1_rev, b1_rev, W1_cc, b1_cc,
           W2_gg, b2_gg, W2_rev, b2_rev, W2_cc, b2_cc,
           edge_index_gg, edge_index_gg_rev, edge_index_cc, edge_label_index):
    x = jnp.pad(x_gene, ((0, NPAD - N), (0, 0)))
    src_gg = _tile_edges(edge_index_gg[0], NS, NCHUNK)
    dst_gg = _tile_edges(edge_index_gg[1], NS, NCHUNK)
    src_rev = _tile_edges(edge_index_gg_rev[0], NS, NCHUNK)
    dst_rev = _tile_edges(edge_index_gg_rev[1], NS, NCHUNK)
    src_gg2 = _tile_edges(edge_index_gg[0], NS * NC, NCHUNK2)
    dst_gg2 = _tile_edges(edge_index_gg[1], NS * NC, NCHUNK2)
    src_rev2 = _tile_edges(edge_index_gg_rev[0], NS * NC, NCHUNK2)
    dst_rev2 = _tile_edges(edge_index_gg_rev[1], NS * NC, NCHUNK2)
    l0 = _tile_edges(edge_label_index[0], NS * NC, NCHUNK_L)
    l1 = _tile_edges(edge_label_index[1], NS * NC, NCHUNK_L)

    zeros8 = jnp.zeros((8, NPAD), jnp.float32)
    z128 = jnp.zeros((NPAD, H1 // 2), jnp.float32)

    dst_cat = jnp.concatenate([dst_gg, dst_rev], axis=0).reshape(NC * NS, -1)
    hist = _degrees(dst_cat, zeros8).reshape(NC, NS * 8, NPAD)
    dinv_gg, dinv_rev = _tc0(hist[0], hist[1])
    dinv_gg = dinv_gg.reshape(NPAD, 1)
    dinv_rev = dinv_rev.reshape(NPAD, 1)

    ygg_lo, ygg_hi, yrev_lo, yrev_hi = _tc1(x, W1_gg, W1_rev, dinv_gg, dinv_rev)

    src_gg_cat = jnp.concatenate([src_gg, src_gg + NPAD], axis=0)
    src_rev_cat = jnp.concatenate([src_rev, src_rev + NPAD], axis=0)
    ygg_cat = jnp.concatenate([ygg_lo, ygg_hi], axis=0)
    yrev_cat = jnp.concatenate([yrev_lo, yrev_hi], axis=0)
    agg = _scatter_h1(ygg_cat, yrev_cat, _blk4(src_gg_cat), _blk4(src_rev_cat),
                      _blk4(dst_gg), _blk4(dst_rev), z128)

    y2gg, y2rev = _tc2(
        agg[0, 0], agg[0, 1], agg[1, 0], agg[1, 1],
        ygg_lo, ygg_hi, yrev_lo, yrev_hi, dinv_gg, dinv_rev,
        b1_gg.reshape(1, H1), b1_rev.reshape(1, H1), W2_gg, W2_rev)

    agg2 = _scatter_h2(y2gg, y2rev, _blk4(src_gg2), _blk4(src_rev2),
                       _blk4(dst_gg2), _blk4(dst_rev2), z128)

    g2 = _tc3(agg2[0, 0], agg2[0, 1], agg2[1, 0], agg2[1, 1],
              y2gg, y2rev, dinv_gg, dinv_rev,
              b2_gg.reshape(1, H2), b2_rev.reshape(1, H2))

    pred16 = _label_dot(g2, l0, l1)
    sel = (jnp.arange(K)[:, None] // 16 == jnp.arange(8)[None, :]).astype(jnp.float32)
    pred = _tc4(pred16, sel)
    return pred.reshape(EPAD_L)[:E_LBL]
